# Initial kernel scaffold; baseline (speedup 1.0000x reference)
import jax
import jax.numpy as jnp
from jax.experimental import pallas as pl


def kernel(locs, data, neighbors, qlocs, weight, bias):
    raise NotImplementedError("write your pallas kernel here")


# speedup vs baseline: 3.2177x; 3.2177x over previous
import functools

import numpy as np
import jax
import jax.numpy as jnp
from jax import lax
from jax.experimental import pallas as pl
from jax.experimental.pallas import tpu as pltpu
from jax.experimental.pallas import tpu_sc as plsc

RADIUS = 0.1
KERNEL_SIZE = 3
DILATION = 0.05
NDIM = 3
NCELLS = KERNEL_SIZE ** NDIM


def _cell_offsets_np():
    half = (KERNEL_SIZE - 1) / 2.0
    ax = (np.arange(KERNEL_SIZE) - half) * DILATION
    grids = np.meshgrid(*([ax] * NDIM), indexing='ij')
    return np.stack([g.reshape(-1) for g in grids], axis=-1).astype(np.float32)


_OFFS = _cell_offsets_np()


def _make_sc_gather(n_rows_tab, n_idx, width):
    info = plsc.get_sparse_core_info()
    nc, ns = info.num_cores, info.num_subcores
    nw = nc * ns
    ch = 128
    assert n_idx % (nw * ch) == 0
    per_w = n_idx // nw
    iters = per_w // ch
    mesh = plsc.VectorSubcoreMesh(core_axis_name="c", subcore_axis_name="s")

    @functools.partial(
        pl.kernel,
        mesh=mesh,
        out_type=jax.ShapeDtypeStruct((n_idx, width), jnp.float32),
        scratch_types=[
            pltpu.VMEM((ch,), jnp.int32),
            pltpu.VMEM((ch, width), jnp.float32),
            pltpu.SemaphoreType.DMA,
        ],
        compiler_params=pltpu.CompilerParams(use_tc_tiling_on_sc=False),
    )
    def gather(table_hbm, idx_hbm, out_hbm, idx_v, rows_v, sem):
        wid = lax.axis_index("s") * nc + lax.axis_index("c")
        base = wid * per_w

        def body(i, carry):
            off = base + i * ch
            pltpu.sync_copy(idx_hbm.at[pl.ds(off, ch)], idx_v)
            pltpu.async_copy(table_hbm.at[idx_v], rows_v, sem).wait()
            pltpu.sync_copy(rows_v, out_hbm.at[pl.ds(off, ch)])
            return carry

        lax.fori_loop(0, iters, body, 0)

    return gather


def _tc_body(k, c, mb, g_ref, q_ref, w2_ref, b_ref, o_ref):
    norm = 1.0 / (RADIUS ** 3)
    half = (KERNEL_SIZE - 1) / 2.0
    ci = lax.broadcasted_iota(jnp.int32, (1, NCELLS), 1)
    ox = ((ci // (KERNEL_SIZE * KERNEL_SIZE)).astype(jnp.float32) - half) * DILATION
    oy = (((ci // KERNEL_SIZE) % KERNEL_SIZE).astype(jnp.float32) - half) * DILATION
    oz = ((ci % KERNEL_SIZE).astype(jnp.float32) - half) * DILATION

    g = g_ref[...]
    dg = g[:, 0:c]
    d0 = q_ref[:, 0:1] - g[:, c:c + 1]
    d1 = q_ref[:, 1:2] - g[:, c + 1:c + 2]
    d2 = q_ref[:, 2:3] - g[:, c + 2:c + 3]
    e0 = d0 + ox
    e1 = d1 + oy
    e2 = d2 + oz
    dist2 = e0 * e0 + e1 * e1 + e2 * e2
    dist = jnp.sqrt(dist2 + 1e-12)
    t = jnp.maximum(1.0 - dist * (1.0 / RADIUS), 0.0)
    w = norm * t * t * t

    pieces = [w[:, cell:cell + 1] * dg for cell in range(NCELLS)]
    p = jnp.concatenate(pieces, axis=1)
    interp = p.reshape(mb, k, NCELLS * c).sum(axis=1)
    acc = jnp.dot(interp, w2_ref[...], preferred_element_type=jnp.float32,
                  precision=jax.lax.Precision.HIGHEST)
    o_ref[...] = acc + b_ref[...]


def kernel(locs, data, neighbors, qlocs, weight, bias):
    b, n, d = locs.shape
    _, m, k = neighbors.shape
    c = data.shape[2]
    o = weight.shape[0]
    width = 48
    r = b * m * k
    bm = b * m

    feat = jnp.concatenate(
        [data, locs, jnp.zeros((b, n, width - c - d), jnp.float32)], axis=-1)
    table = jnp.concatenate(
        [feat.reshape(b * n, width), jnp.zeros((8, width), jnp.float32)], axis=0)

    nb = neighbors.astype(jnp.int32)
    base = (jnp.arange(b, dtype=jnp.int32) * n)[:, None, None]
    flat_idx = jnp.where(nb < 0, b * n, nb + base).reshape(r)

    q4 = jnp.concatenate([qlocs, jnp.zeros((b, m, 1), jnp.float32)], axis=-1)
    qrep = jnp.broadcast_to(q4[:, :, None, :], (b, m, k, 4)).reshape(r, 4)

    gathered = _make_sc_gather(b * n + 8, r, width)(table, flat_idx)

    w2p = jnp.transpose(weight, (2, 1, 0)).reshape(NCELLS * c, o)
    mb = 64
    out2 = pl.pallas_call(
        functools.partial(_tc_body, k, c, mb),
        grid=(bm // mb,),
        in_specs=[
            pl.BlockSpec((mb * k, width), lambda i: (i, 0)),
            pl.BlockSpec((mb * k, 4), lambda i: (i, 0)),
            pl.BlockSpec((NCELLS * c, o), lambda i: (0, 0)),
            pl.BlockSpec((1, o), lambda i: (0, 0)),
        ],
        out_specs=pl.BlockSpec((mb, o), lambda i: (i, 0)),
        out_shape=jax.ShapeDtypeStruct((bm, o), jnp.float32),
    )(gathered, qrep, w2p, bias.reshape(1, o))

    return out2.reshape(b, m, o)


# speedup vs baseline: 8.5694x; 2.6632x over previous
import functools

import numpy as np
import jax
import jax.numpy as jnp
from jax import lax
from jax.experimental import pallas as pl
from jax.experimental.pallas import tpu as pltpu
from jax.experimental.pallas import tpu_sc as plsc

RADIUS = 0.1
KERNEL_SIZE = 3
DILATION = 0.05
NDIM = 3
NCELLS = KERNEL_SIZE ** NDIM


def _cell_offsets_np():
    half = (KERNEL_SIZE - 1) / 2.0
    ax = (np.arange(KERNEL_SIZE) - half) * DILATION
    grids = np.meshgrid(*([ax] * NDIM), indexing='ij')
    return np.stack([g.reshape(-1) for g in grids], axis=-1).astype(np.float32)


_OFFS = _cell_offsets_np()


def _make_sc_gather(n_rows_tab, n_idx, width):
    info = plsc.get_sparse_core_info()
    nc, ns = info.num_cores, info.num_subcores
    nw = nc * ns
    ch = 128
    assert n_idx % (nw * ch) == 0
    per_w = n_idx // nw
    iters = per_w // ch
    mesh = plsc.VectorSubcoreMesh(core_axis_name="c", subcore_axis_name="s")

    @functools.partial(
        pl.kernel,
        mesh=mesh,
        out_type=jax.ShapeDtypeStruct((n_idx, width), jnp.float32),
        scratch_types=[
            pltpu.VMEM((ch,), jnp.int32),
            pltpu.VMEM((ch, width), jnp.float32),
            pltpu.SemaphoreType.DMA,
        ],
        compiler_params=pltpu.CompilerParams(use_tc_tiling_on_sc=False),
    )
    def gather(table_hbm, idx_hbm, out_hbm, idx_v, rows_v, sem):
        wid = lax.axis_index("s") * nc + lax.axis_index("c")
        base = wid * per_w

        def body(i, carry):
            off = base + i * ch
            pltpu.sync_copy(idx_hbm.at[pl.ds(off, ch)], idx_v)
            pltpu.async_copy(table_hbm.at[idx_v], rows_v, sem).wait()
            pltpu.sync_copy(rows_v, out_hbm.at[pl.ds(off, ch)])
            return carry

        lax.fori_loop(0, iters, body, 0)

    return gather


def _tc_body(k, c, mb, g_ref, q_ref, w2_ref, b_ref, o_ref):
    norm = 1.0 / (RADIUS ** 3)
    half = (KERNEL_SIZE - 1) / 2.0
    ci = lax.broadcasted_iota(jnp.int32, (1, 32), 1)
    ox = ((ci // (KERNEL_SIZE * KERNEL_SIZE)).astype(jnp.float32) - half) * DILATION
    oy = (((ci // KERNEL_SIZE) % KERNEL_SIZE).astype(jnp.float32) - half) * DILATION
    oz = ((ci % KERNEL_SIZE).astype(jnp.float32) - half) * DILATION

    g = g_ref[...]
    dg = g[:, 0:c]
    d0 = q_ref[:, 0:1] - g[:, c:c + 1]
    d1 = q_ref[:, 1:2] - g[:, c + 1:c + 2]
    d2 = q_ref[:, 2:3] - g[:, c + 2:c + 3]
    e0 = d0 + ox
    e1 = d1 + oy
    e2 = d2 + oz
    dist2 = e0 * e0 + e1 * e1 + e2 * e2
    dist = jnp.sqrt(dist2 + 1e-12)
    t = jnp.maximum(1.0 - dist * (1.0 / RADIUS), 0.0)
    w = norm * t * t * t

    nc32 = w.shape[1]
    wide = NCELLS * c
    widep = 128 * ((wide + 127) // 128)
    row_i = lax.broadcasted_iota(jnp.int32, (nc32, widep), 0)
    col_i = lax.broadcasted_iota(jnp.int32, (nc32, widep), 1)
    rep = jnp.where((col_i // c == row_i) & (col_i < wide), 1.0, 0.0)
    rep = rep.astype(jnp.float32)

    hi = w.astype(jnp.bfloat16).astype(jnp.float32)
    lo = w - hi
    wexp = (jnp.dot(hi, rep, preferred_element_type=jnp.float32)
            + jnp.dot(lo, rep, preferred_element_type=jnp.float32))

    dg4 = jnp.concatenate([dg, dg, dg, dg], axis=1)
    dgt = jnp.concatenate([dg4] * (widep // 128), axis=1)

    p = wexp * dgt
    interp = p.reshape(mb, k, widep).sum(axis=1)
    acc = jnp.dot(interp, w2_ref[...], preferred_element_type=jnp.float32,
                  precision=jax.lax.Precision.HIGHEST)
    o_ref[...] = acc + b_ref[...]


def kernel(locs, data, neighbors, qlocs, weight, bias):
    b, n, d = locs.shape
    _, m, k = neighbors.shape
    c = data.shape[2]
    o = weight.shape[0]
    width = 48
    r = b * m * k
    bm = b * m

    feat = jnp.concatenate(
        [data, locs, jnp.zeros((b, n, width - c - d), jnp.float32)], axis=-1)
    table = jnp.concatenate(
        [feat.reshape(b * n, width), jnp.zeros((8, width), jnp.float32)], axis=0)

    nb = neighbors.astype(jnp.int32)
    base = (jnp.arange(b, dtype=jnp.int32) * n)[:, None, None]
    flat_idx = jnp.where(nb < 0, b * n, nb + base).reshape(r)

    q4 = jnp.concatenate([qlocs, jnp.zeros((b, m, 1), jnp.float32)], axis=-1)
    qrep = jnp.broadcast_to(q4[:, :, None, :], (b, m, k, 4)).reshape(r, 4)

    gathered = _make_sc_gather(b * n + 8, r, width)(table, flat_idx)

    w2p = jnp.transpose(weight, (2, 1, 0)).reshape(NCELLS * c, o)
    widep = 128 * ((NCELLS * c + 127) // 128)
    w2p = jnp.concatenate(
        [w2p, jnp.zeros((widep - NCELLS * c, o), jnp.float32)], axis=0)
    mb = 128
    out2 = pl.pallas_call(
        functools.partial(_tc_body, k, c, mb),
        grid=(bm // mb,),
        in_specs=[
            pl.BlockSpec((mb * k, width), lambda i: (i, 0)),
            pl.BlockSpec((mb * k, 4), lambda i: (i, 0)),
            pl.BlockSpec((widep, o), lambda i: (0, 0)),
            pl.BlockSpec((1, o), lambda i: (0, 0)),
        ],
        out_specs=pl.BlockSpec((mb, o), lambda i: (i, 0)),
        out_shape=jax.ShapeDtypeStruct((bm, o), jnp.float32),
    )(gathered, qrep, w2p, bias.reshape(1, o))

    return out2.reshape(b, m, o)
